# baseline (speedup 1.0000x reference)
import jax
import jax.numpy as jnp
from jax.experimental import pallas as pl


def _tc_body(x_ref, bd_ref, bwide_ref, r0w_ref, out_ref):
    bn, l = out_ref.shape[1], out_ref.shape[2]
    out_ref[0] = jnp.broadcast_to(r0w_ref[...], (bn, l))
    out_ref[1] = (
        jnp.dot(x_ref[0], bd_ref[...], preferred_element_type=jnp.float32)
        + bwide_ref[...]
    )


def kernel(x, token, W, b):
    B, N, C = x.shape
    K = W.shape[0]
    PACK = 128 // C
    NW = N // PACK
    Wt = W.T
    bd = jnp.kron(jnp.eye(PACK, dtype=x.dtype), Wt)
    bwide = jnp.tile(b, PACK).reshape(1, PACK * K)
    r0 = token.reshape(1, C) @ Wt + b.reshape(1, K)
    r0w = jnp.tile(r0, PACK)
    xw = x.reshape(B, NW, PACK * C)

    Bn = min(4096, NW)
    grid = (NW // Bn,)
    outw = pl.pallas_call(
        _tc_body,
        grid=grid,
        in_specs=[
            pl.BlockSpec((1, Bn, PACK * C), lambda i: (1, i, 0)),
            pl.BlockSpec((PACK * C, PACK * K), lambda i: (0, 0)),
            pl.BlockSpec((1, PACK * K), lambda i: (0, 0)),
            pl.BlockSpec((1, PACK * K), lambda i: (0, 0)),
        ],
        out_specs=pl.BlockSpec((2, Bn, PACK * K), lambda i: (0, i, 0)),
        out_shape=jax.ShapeDtypeStruct((B, NW, PACK * K), x.dtype),
    )(xw, bd, bwide, r0w)
    return outw.reshape(B, N, K)


# speedup vs baseline: 1.1044x; 1.1044x over previous
import jax
import jax.numpy as jnp
from jax.experimental import pallas as pl


def _tc_body(x_ref, wt_ref, b_ref, r0_ref, out_ref):
    bn, k = out_ref.shape[1], out_ref.shape[2]
    out_ref[0] = jnp.broadcast_to(r0_ref[...], (bn, k))
    out_ref[1] = (
        jnp.dot(x_ref[0], wt_ref[...], preferred_element_type=jnp.float32)
        + b_ref[...]
    )


def kernel(x, token, W, b):
    B, N, C = x.shape
    K = W.shape[0]
    Wt = W.T
    b2 = b.reshape(1, K)
    r0 = token.reshape(1, C) @ Wt + b2

    Bn = min(8192, N)
    grid = (N // Bn,)
    return pl.pallas_call(
        _tc_body,
        grid=grid,
        in_specs=[
            pl.BlockSpec((1, Bn, C), lambda i: (1, i, 0)),
            pl.BlockSpec((C, K), lambda i: (0, 0)),
            pl.BlockSpec((1, K), lambda i: (0, 0)),
            pl.BlockSpec((1, K), lambda i: (0, 0)),
        ],
        out_specs=pl.BlockSpec((2, Bn, K), lambda i: (0, i, 0)),
        out_shape=jax.ShapeDtypeStruct((B, N, K), x.dtype),
    )(x, Wt, b2, r0)


# speedup vs baseline: 31.0828x; 28.1451x over previous
import jax
import jax.numpy as jnp
from jax.experimental import pallas as pl


def _tc_body(x_ref, w_ref, b_ref, r0_ref, out_ref):
    k, bn = out_ref.shape[1], out_ref.shape[2]
    out_ref[0] = jnp.broadcast_to(r0_ref[...], (k, bn))
    out_ref[1] = (
        jnp.dot(w_ref[...], x_ref[0], preferred_element_type=jnp.float32)
        + b_ref[...]
    )


def kernel(x, token, W, b):
    B, N, C = x.shape
    K = W.shape[0]
    b_col = b.reshape(K, 1)
    r0_col = W @ token.reshape(C, 1) + b_col
    xt = jnp.transpose(x, (0, 2, 1))

    Bn = min(65536, N)
    grid = (N // Bn,)
    out_t = pl.pallas_call(
        _tc_body,
        grid=grid,
        in_specs=[
            pl.BlockSpec((1, C, Bn), lambda i: (1, 0, i)),
            pl.BlockSpec((K, C), lambda i: (0, 0)),
            pl.BlockSpec((K, 1), lambda i: (0, 0)),
            pl.BlockSpec((K, 1), lambda i: (0, 0)),
        ],
        out_specs=pl.BlockSpec((2, K, Bn), lambda i: (0, 0, i)),
        out_shape=jax.ShapeDtypeStruct((B, K, N), x.dtype),
    )(xt, W, b_col, r0_col)
    return jnp.transpose(out_t, (0, 2, 1))


# speedup vs baseline: 31.8618x; 1.0251x over previous
import jax
import jax.numpy as jnp
from jax.experimental import pallas as pl


def _tc_body(x_ref, w_ref, b_ref, r0_ref, out_ref):
    k, bn = out_ref.shape[1], out_ref.shape[2]
    out_ref[0] = jnp.broadcast_to(r0_ref[...], (k, bn))
    out_ref[1] = (
        jnp.dot(w_ref[...], x_ref[0], preferred_element_type=jnp.float32)
        + b_ref[...]
    )


def kernel(x, token, W, b):
    B, N, C = x.shape
    K = W.shape[0]
    b_col = b.reshape(K, 1)
    r0_col = W @ token.reshape(C, 1) + b_col
    xt = jnp.transpose(x, (0, 2, 1))

    Bn = min(131072, N)
    grid = (N // Bn,)
    out_t = pl.pallas_call(
        _tc_body,
        grid=grid,
        in_specs=[
            pl.BlockSpec((1, C, Bn), lambda i: (1, 0, i)),
            pl.BlockSpec((K, C), lambda i: (0, 0)),
            pl.BlockSpec((K, 1), lambda i: (0, 0)),
            pl.BlockSpec((K, 1), lambda i: (0, 0)),
        ],
        out_specs=pl.BlockSpec((2, K, Bn), lambda i: (0, 0, i)),
        out_shape=jax.ShapeDtypeStruct((B, K, N), x.dtype),
    )(xt, W, b_col, r0_col)
    return jnp.transpose(out_t, (0, 2, 1))
